# baseline (speedup 1.0000x reference)
import jax
import jax.numpy as jnp
from jax.experimental import pallas as pl
from jax.experimental.pallas import tpu as pltpu

B, S, D, N = 128, 512, 512, 20
NP = 24
BBG = 8
MB = 32
SS = 128
NBI = B // MB
NS = S // SS


def _agg_pair(node, order):
    maskf = (order > 0).astype(jnp.float32)
    ii = jax.lax.broadcasted_iota(jnp.int32, order.shape, 1)

    gtc = (order[:, None, :] > order[:, :, None]).astype(jnp.float32)
    cnt_gt = jnp.sum(gtc * maskf[:, None, :], axis=2)
    m_tot = jnp.sum(maskf, axis=1, keepdims=True)
    dG = 1.0 + maskf * cnt_gt
    dL = 2.0 + maskf * (m_tot - maskf) - dG

    masked_node = node * maskf[:, :, None]
    t_sum = jnp.sum(masked_node, axis=1, keepdims=True)
    agg_sum = maskf[:, :, None] * (t_sum - masked_node) + 2.0 * node

    aG = jnp.zeros(node.shape, jnp.float32)
    for j in range(N):
        oj = order[:, j:j + 1]
        mj = maskf[:, j:j + 1]
        cG = maskf * mj * (order > oj).astype(jnp.float32)
        cG = cG + (ii == j).astype(jnp.float32)
        aG = aG + cG[:, :, None] * node[:, j:j + 1, :]
    aL = agg_sum - aG
    return aG / dG[:, :, None], aL / dL[:, :, None]


def _merged_body(pos_ref, node_ref, order_ref,
                 w1t0_ref, b10_ref, w2t0_ref, b20_ref, wot0_ref, bo0_ref,
                 w1t1_ref, b11_ref, w2t1_ref, b21_ref, wot1_ref, bo1_ref,
                 enc_ref, out_ref, embout_ref, pmax_ref, embbuf_ref):
    i = pl.program_id(0)
    s = pl.program_id(1)

    @pl.when(i * NS + s < NBI * NS)
    def gnn_phase():
        node0 = jnp.concatenate(
            [node_ref[...], jnp.zeros((BBG, NP - N, D), jnp.float32)], axis=1)
        order = jnp.concatenate(
            [order_ref[...], jnp.zeros((BBG, NP - N), jnp.int32)], axis=1)
        node = node0
        hops = ((w1t0_ref, b10_ref, w2t0_ref, b20_ref, wot0_ref, bo0_ref),
                (w1t1_ref, b11_ref, w2t1_ref, b21_ref, wot1_ref, bo1_ref))
        for (w1t_ref, b1_ref, w2t_ref, b2_ref, wot_ref, bo_ref) in hops:
            aG, aL = _agg_pair(node, order)
            xG = aG.reshape(BBG * NP, D)
            xL = aL.reshape(BBG * NP, D)
            dnt = (((1,), (1,)), ((), ()))
            n1 = jax.nn.relu(
                jax.lax.dot_general(xG, w1t_ref[...], dnt,
                                    preferred_element_type=jnp.float32)
                + b1_ref[...])
            n2 = jax.nn.relu(
                jax.lax.dot_general(xL, w2t_ref[...], dnt,
                                    preferred_element_type=jnp.float32)
                + b2_ref[...])
            wot = wot_ref[...]
            out = jax.nn.relu(
                jax.lax.dot_general(n1, wot[:, :D], dnt,
                                    preferred_element_type=jnp.float32)
                + jax.lax.dot_general(n2, wot[:, D:], dnt,
                                      preferred_element_type=jnp.float32)
                + bo_ref[...])
            node = out.reshape(BBG, NP, D)
        slot = jax.lax.rem(i, 2)
        embbuf_ref[pl.ds(slot, 1), pl.ds(s * BBG, BBG)] = node[None]
        embout_ref[...] = node0[:, :N, :] + node[:, :N, :]

    @pl.when(i > 0)
    def fuse_phase():
        out_ref[...] = enc_ref[...]
        slotr = jax.lax.rem(i + 1, 2)
        for b in range(MB):
            for n in range(N):
                idx = pos_ref[b, n]
                lidx = idx - s * SS

                @pl.when((lidx >= 0) & (lidx < SS))
                def _scatter():
                    out_ref[pl.ds(lidx, 1), b, :] = (
                        enc_ref[pl.ds(lidx, 1), b, :]
                        + embbuf_ref[slotr, b, n, :][None, :])
        partial = jnp.max(out_ref[...], axis=0)

        @pl.when(s == 0)
        def _init():
            pmax_ref[...] = partial

        @pl.when(s > 0)
        def _fold():
            pmax_ref[...] = jnp.maximum(pmax_ref[...], partial)


def kernel(encoder_outputs, num_encoder_outputs, num_pos_pad, num_order_pad,
           fc1_w_0, fc1_b_0, fc2_w_0, fc2_b_0, out_w_0, out_b_0,
           fc1_w_1, fc1_b_1, fc2_w_1, fc2_b_1, out_w_1, out_b_1):
    f32 = jnp.float32
    ilag = lambda i: jnp.maximum(i - 1, 0)
    iksub = lambda i, s: jnp.minimum(i * NS + s, NBI * NS - 1)
    senc = lambda i, s: jnp.where(i > 0, s, 0)
    wspec = lambda shp: pl.BlockSpec(shp, lambda i, s: (0,) * len(shp))
    in_specs = [
        pl.BlockSpec((MB, N), lambda i, s: (ilag(i), 0),
                     memory_space=pltpu.SMEM),
        pl.BlockSpec((BBG, N, D), lambda i, s: (iksub(i, s), 0, 0)),
        pl.BlockSpec((BBG, N), lambda i, s: (iksub(i, s), 0)),
    ]
    weights = []
    for (w1, b1, w2, b2, wo, bo) in ((fc1_w_0, fc1_b_0, fc2_w_0, fc2_b_0, out_w_0, out_b_0),
                                     (fc1_w_1, fc1_b_1, fc2_w_1, fc2_b_1, out_w_1, out_b_1)):
        weights += [w1, b1.reshape(1, D), w2, b2.reshape(1, D),
                    wo, bo.reshape(1, D)]
        in_specs += [wspec((D, D)), wspec((1, D)), wspec((D, D)),
                     wspec((1, D)), wspec((D, 2 * D)), wspec((1, D))]
    in_specs.append(
        pl.BlockSpec((SS, MB, D), lambda i, s: (senc(i, s), ilag(i), 0)))

    out, embout, pmax = pl.pallas_call(
        _merged_body,
        grid=(NBI + 1, NS),
        in_specs=in_specs,
        out_specs=[
            pl.BlockSpec((SS, MB, D), lambda i, s: (senc(i, s), ilag(i), 0)),
            pl.BlockSpec((BBG, N, D), lambda i, s: (iksub(i, s), 0, 0)),
            pl.BlockSpec((MB, D), lambda i, s: (ilag(i), 0)),
        ],
        out_shape=[jax.ShapeDtypeStruct((S, B, D), f32),
                   jax.ShapeDtypeStruct((B, N, D), f32),
                   jax.ShapeDtypeStruct((B, D), f32)],
        scratch_shapes=[pltpu.VMEM((2, MB, NP, D), f32)],
    )(num_pos_pad, num_encoder_outputs, num_order_pad, *weights,
      encoder_outputs)

    return out, embout, pmax


# speedup vs baseline: 1.6663x; 1.6663x over previous
import jax
import jax.numpy as jnp
from jax.experimental import pallas as pl
from jax.experimental.pallas import tpu as pltpu

B, S, D, N = 128, 512, 512, 20
NP = 24


def _agg_pair(node, order):
    maskf = (order > 0).astype(jnp.float32)
    ii = jax.lax.broadcasted_iota(jnp.int32, order.shape, 1)

    gtc = (order[:, None, :] > order[:, :, None]).astype(jnp.float32)
    cnt_gt = jnp.sum(gtc * maskf[:, None, :], axis=2)
    m_tot = jnp.sum(maskf, axis=1, keepdims=True)
    dG = 1.0 + maskf * cnt_gt
    dL = 2.0 + maskf * (m_tot - maskf) - dG

    masked_node = node * maskf[:, :, None]
    t_sum = jnp.sum(masked_node, axis=1, keepdims=True)
    agg_sum = maskf[:, :, None] * (t_sum - masked_node) + 2.0 * node

    aG = jnp.zeros(node.shape, jnp.float32)
    for j in range(N):
        oj = order[:, j:j + 1]
        mj = maskf[:, j:j + 1]
        cG = maskf * mj * (order > oj).astype(jnp.float32)
        cG = cG + (ii == j).astype(jnp.float32)
        aG = aG + cG[:, :, None] * node[:, j:j + 1, :]
    aL = agg_sum - aG
    return aG / dG[:, :, None], aL / dL[:, :, None]


def _merged_body(pos_ref, node_ref, order_ref,
                 w1t0_ref, w2t0_ref, wot0_ref,
                 w1t1_ref, w2t1_ref, wot1_ref, bmat_ref,
                 enc_ref, out_ref, embout_ref, pmax_ref, embbuf_ref,
                 bb, nblk):
    i = pl.program_id(0)

    @pl.when(i < nblk)
    def gnn_phase():
        node0 = jnp.concatenate(
            [node_ref[...], jnp.zeros((bb, NP - N, D), jnp.float32)], axis=1)
        order = jnp.concatenate(
            [order_ref[...], jnp.zeros((bb, NP - N), jnp.int32)], axis=1)
        node = node0
        bmat = bmat_ref[...]
        hops = ((w1t0_ref, bmat[0:1], w2t0_ref, bmat[1:2], wot0_ref, bmat[2:3]),
                (w1t1_ref, bmat[3:4], w2t1_ref, bmat[4:5], wot1_ref, bmat[5:6]))
        for (w1t_ref, b1, w2t_ref, b2, wot_ref, bo) in hops:
            aG, aL = _agg_pair(node, order)
            xG = aG.reshape(bb * NP, D)
            xL = aL.reshape(bb * NP, D)
            dnt = (((1,), (1,)), ((), ()))
            n1 = jax.nn.relu(
                jax.lax.dot_general(xG, w1t_ref[...], dnt,
                                    preferred_element_type=jnp.float32)
                + b1)
            n2 = jax.nn.relu(
                jax.lax.dot_general(xL, w2t_ref[...], dnt,
                                    preferred_element_type=jnp.float32)
                + b2)
            wot = wot_ref[...]
            out = jax.nn.relu(
                jax.lax.dot_general(n1, wot[:, :D], dnt,
                                    preferred_element_type=jnp.float32)
                + jax.lax.dot_general(n2, wot[:, D:], dnt,
                                      preferred_element_type=jnp.float32)
                + bo)
            node = out.reshape(bb, NP, D)
        embbuf_ref[pl.ds(jax.lax.rem(i, 2), 1)] = node[None]
        embout_ref[...] = node0[:, :N, :] + node[:, :N, :]

    @pl.when(i > 0)
    def fuse_phase():
        out_ref[...] = enc_ref[...]
        slot = jax.lax.rem(i + 1, 2)
        for b in range(bb):
            for n in range(N):
                idx = pos_ref[b, n]
                row = embbuf_ref[slot, b, n, :]
                out_ref[pl.ds(idx, 1), b, :] = (
                    enc_ref[pl.ds(idx, 1), b, :] + row[None, :])
        pmax_ref[...] = jnp.max(out_ref[...], axis=0)


def kernel(encoder_outputs, num_encoder_outputs, num_pos_pad, num_order_pad,
           fc1_w_0, fc1_b_0, fc2_w_0, fc2_b_0, out_w_0, out_b_0,
           fc1_w_1, fc1_b_1, fc2_w_1, fc2_b_1, out_w_1, out_b_1):
    f32 = jnp.float32

    BB = 8
    NBLK = B // BB
    ilag = lambda i: jnp.maximum(i - 1, 0)
    icur = lambda i: jnp.minimum(i, NBLK - 1)
    wspec = lambda shp: pl.BlockSpec(shp, lambda i: (0,) * len(shp))
    in_specs = [
        pl.BlockSpec((BB, N), lambda i: (ilag(i), 0),
                     memory_space=pltpu.SMEM),
        pl.BlockSpec((BB, N, D), lambda i: (icur(i), 0, 0)),
        pl.BlockSpec((BB, N), lambda i: (icur(i), 0)),
    ]
    weights = [fc1_w_0, fc2_w_0, out_w_0, fc1_w_1, fc2_w_1, out_w_1,
               jnp.stack([fc1_b_0, fc2_b_0, out_b_0,
                          fc1_b_1, fc2_b_1, out_b_1])]
    in_specs += [wspec((D, D)), wspec((D, D)), wspec((D, 2 * D)),
                 wspec((D, D)), wspec((D, D)), wspec((D, 2 * D)),
                 wspec((6, D))]
    in_specs.append(pl.BlockSpec((S, BB, D), lambda i: (0, ilag(i), 0)))

    out, embout, pmax = pl.pallas_call(
        lambda *refs: _merged_body(*refs, BB, NBLK),
        grid=(NBLK + 1,),
        in_specs=in_specs,
        out_specs=[
            pl.BlockSpec((S, BB, D), lambda i: (0, ilag(i), 0)),
            pl.BlockSpec((BB, N, D), lambda i: (icur(i), 0, 0)),
            pl.BlockSpec((BB, D), lambda i: (ilag(i), 0)),
        ],
        out_shape=[jax.ShapeDtypeStruct((S, B, D), f32),
                   jax.ShapeDtypeStruct((B, N, D), f32),
                   jax.ShapeDtypeStruct((B, D), f32)],
        scratch_shapes=[pltpu.VMEM((2, BB, NP, D), f32)],
    )(num_pos_pad, num_encoder_outputs, num_order_pad, *weights,
      encoder_outputs)

    return out, embout, pmax


# speedup vs baseline: 1.6876x; 1.0128x over previous
import jax
import jax.numpy as jnp
from jax.experimental import pallas as pl
from jax.experimental.pallas import tpu as pltpu

B, S, D, N = 128, 512, 512, 20
NP = 24


def _agg_pair(node, order):
    maskf = (order > 0).astype(jnp.float32)
    ii = jax.lax.broadcasted_iota(jnp.int32, order.shape, 1)

    gtc = (order[:, None, :] > order[:, :, None]).astype(jnp.float32)
    cnt_gt = jnp.sum(gtc * maskf[:, None, :], axis=2)
    m_tot = jnp.sum(maskf, axis=1, keepdims=True)
    dG = 1.0 + maskf * cnt_gt
    dL = 2.0 + maskf * (m_tot - maskf) - dG

    masked_node = node * maskf[:, :, None]
    t_sum = jnp.sum(masked_node, axis=1, keepdims=True)
    agg_sum = maskf[:, :, None] * (t_sum - masked_node) + 2.0 * node

    aG = jnp.zeros(node.shape, jnp.float32)
    for j in range(N):
        oj = order[:, j:j + 1]
        mj = maskf[:, j:j + 1]
        cG = maskf * mj * (order > oj).astype(jnp.float32)
        cG = cG + (ii == j).astype(jnp.float32)
        aG = aG + cG[:, :, None] * node[:, j:j + 1, :]
    aL = agg_sum - aG
    return aG / dG[:, :, None], aL / dL[:, :, None]


def _merged_body(pos_ref, node_ref, order_ref,
                 w1t0_ref, b10_ref, w2t0_ref, b20_ref, wot0_ref, bo0_ref,
                 w1t1_ref, b11_ref, w2t1_ref, b21_ref, wot1_ref, bo1_ref,
                 enc_ref, out_ref, embout_ref, pmax_ref, embbuf_ref,
                 bb, nblk):
    i = pl.program_id(0)

    @pl.when(i < nblk)
    def gnn_phase():
        node0 = jnp.concatenate(
            [node_ref[...], jnp.zeros((bb, NP - N, D), jnp.float32)], axis=1)
        order = jnp.concatenate(
            [order_ref[...], jnp.zeros((bb, NP - N), jnp.int32)], axis=1)
        node = node0
        hops = ((w1t0_ref, b10_ref, w2t0_ref, b20_ref, wot0_ref, bo0_ref),
                (w1t1_ref, b11_ref, w2t1_ref, b21_ref, wot1_ref, bo1_ref))
        for (w1t_ref, b1_ref, w2t_ref, b2_ref, wot_ref, bo_ref) in hops:
            aG, aL = _agg_pair(node, order)
            xG = aG.reshape(bb * NP, D)
            xL = aL.reshape(bb * NP, D)
            dnt = (((1,), (1,)), ((), ()))
            n1 = jax.nn.relu(
                jax.lax.dot_general(xG, w1t_ref[...], dnt,
                                    preferred_element_type=jnp.float32)
                + b1_ref[...])
            n2 = jax.nn.relu(
                jax.lax.dot_general(xL, w2t_ref[...], dnt,
                                    preferred_element_type=jnp.float32)
                + b2_ref[...])
            wot = wot_ref[...]
            out = jax.nn.relu(
                jax.lax.dot_general(n1, wot[:, :D], dnt,
                                    preferred_element_type=jnp.float32)
                + jax.lax.dot_general(n2, wot[:, D:], dnt,
                                      preferred_element_type=jnp.float32)
                + bo_ref[...])
            node = out.reshape(bb, NP, D)
        embbuf_ref[pl.ds(jax.lax.rem(i, 2), 1)] = node[None]
        embout_ref[...] = node0[:, :N, :] + node[:, :N, :]

    @pl.when(i > 0)
    def fuse_phase():
        out_ref[...] = enc_ref[...]
        slot = jax.lax.rem(i + 1, 2)
        for b in range(bb):
            for n in range(N):
                idx = pos_ref[b, n]
                row = embbuf_ref[slot, b, n, :]
                out_ref[pl.ds(idx, 1), b, :] = (
                    enc_ref[pl.ds(idx, 1), b, :] + row[None, :])
        pmax_ref[...] = jnp.max(out_ref[...], axis=0)


def kernel(encoder_outputs, num_encoder_outputs, num_pos_pad, num_order_pad,
           fc1_w_0, fc1_b_0, fc2_w_0, fc2_b_0, out_w_0, out_b_0,
           fc1_w_1, fc1_b_1, fc2_w_1, fc2_b_1, out_w_1, out_b_1):
    f32 = jnp.float32

    BB = 8
    NBLK = B // BB
    ilag = lambda i: jnp.maximum(i - 1, 0)
    icur = lambda i: jnp.minimum(i, NBLK - 1)
    wspec = lambda shp: pl.BlockSpec(shp, lambda i: (0,) * len(shp))
    in_specs = [
        pl.BlockSpec((BB, N), lambda i: (ilag(i), 0),
                     memory_space=pltpu.SMEM),
        pl.BlockSpec((BB, N, D), lambda i: (icur(i), 0, 0)),
        pl.BlockSpec((BB, N), lambda i: (icur(i), 0)),
    ]
    weights = []
    for (w1, b1, w2, b2, wo, bo) in ((fc1_w_0, fc1_b_0, fc2_w_0, fc2_b_0, out_w_0, out_b_0),
                                     (fc1_w_1, fc1_b_1, fc2_w_1, fc2_b_1, out_w_1, out_b_1)):
        weights += [w1, b1.reshape(1, D), w2, b2.reshape(1, D),
                    wo, bo.reshape(1, D)]
        in_specs += [wspec((D, D)), wspec((1, D)), wspec((D, D)),
                     wspec((1, D)), wspec((D, 2 * D)), wspec((1, D))]
    in_specs.append(pl.BlockSpec((S, BB, D), lambda i: (0, ilag(i), 0)))

    out, embout, pmax = pl.pallas_call(
        lambda *refs: _merged_body(*refs, BB, NBLK),
        grid=(NBLK + 1,),
        in_specs=in_specs,
        out_specs=[
            pl.BlockSpec((S, BB, D), lambda i: (0, ilag(i), 0)),
            pl.BlockSpec((BB, N, D), lambda i: (icur(i), 0, 0)),
            pl.BlockSpec((BB, D), lambda i: (ilag(i), 0)),
        ],
        out_shape=[jax.ShapeDtypeStruct((S, B, D), f32),
                   jax.ShapeDtypeStruct((B, N, D), f32),
                   jax.ShapeDtypeStruct((B, D), f32)],
        scratch_shapes=[pltpu.VMEM((2, BB, NP, D), f32)],
    )(num_pos_pad, num_encoder_outputs, num_order_pad, *weights,
      encoder_outputs)

    return out, embout, pmax
